# baseline (speedup 1.0000x reference)
import functools
import math

import numpy as np
import jax
import jax.numpy as jnp
from jax import lax
from jax.experimental import pallas as pl
from jax.experimental.pallas import tpu as pltpu
from jax.experimental.pallas import tpu_sc as plsc

NUM_HEADS = 16
HEAD_DIM = 64
NUM_BUCKETS = 320
MAX_DISTANCE = 800
T = 2048
VLEN = 2 * T + 16
VROW = 2 * T
NSHIFT = 8
L = 16

QROWS = NUM_HEADS * T
QCHUNK = 128
ROWS_PER_WORKER = T // 2


def _bucket_vector() -> np.ndarray:
    d = np.arange(VLEN, dtype=np.int64) - (T - 1)
    half = NUM_BUCKETS // 2
    threshold = half // 2
    sign = (d >= 0).astype(np.int64)
    a = np.abs(d)
    log_ratio = np.log(np.clip(a.astype(np.float32), 1.0, None) / np.float32(threshold)) \
        / np.float32(math.log(MAX_DISTANCE / threshold))
    log_pos = np.minimum(
        (np.float32(threshold) + log_ratio * np.float32(half - threshold)).astype(np.int64),
        half - 1)
    b = np.where(a < threshold, a, log_pos) + sign * half
    b = np.clip(b, 0, NUM_BUCKETS - 1).astype(np.int32)
    b[2 * T - 1:] = b[2 * T - 2]
    return b


_BUCKETS = _bucket_vector()


def _sc_body(tab_hbm, b_hbm, g_hbm, out_hbm,
             tabv, bvec, gvec, vvec, stageA, stageB, dsemA, dsemB):
    c = lax.axis_index("c")
    s = lax.axis_index("s")
    h = s
    lanes = jnp.arange(L, dtype=jnp.int32)

    pltpu.sync_copy(tab_hbm, tabv)
    pltpu.sync_copy(b_hbm, bvec)
    pltpu.sync_copy(g_hbm, gvec)
    one_plus_g = plsc.load_gather(gvec, [jnp.full((L,), h, jnp.int32)])

    def v_body(i, _):
        m = i * L
        bidx = bvec[pl.ds(m, L)]
        vvec[pl.ds(m, L)] = plsc.load_gather(tabv, [bidx * NUM_HEADS + h]) * one_plus_g
        return 0

    lax.fori_loop(0, VROW // L, v_body, 0)

    row0 = c * ROWS_PER_WORKER
    TPB = T // 128

    def build_and_issue(stage, sem, rb):
        obase = (T - 1) - rb
        base = lanes + obase

        idxs = tuple(base - k for k in range(NSHIFT))
        vals = tuple(plsc.load_gather(vvec, [idxs[k]]) for k in range(NSHIFT))

        def t_body(t, carry):
            idxs, vals = carry
            new_idxs = tuple(idxs[k] + L for k in range(NSHIFT))
            new_vals = tuple(
                plsc.load_gather(vvec, [new_idxs[k]]) for k in range(NSHIFT))
            row0 = (t // NSHIFT) * NSHIFT
            col = (t % NSHIFT) * L
            for k in range(NSHIFT):
                stage[row0 + k, pl.ds(col, L)] = vals[k]
            return (new_idxs, new_vals)

        lax.fori_loop(0, T // L, t_body, (idxs, vals))

        for ct in range(TPB):
            pltpu.async_copy(
                stage.at[pl.ds(ct * NSHIFT, NSHIFT), :],
                out_hbm.at[0, h, pl.ds(pl.multiple_of(rb, 8), NSHIFT),
                           pl.ds(ct * 128, 128)],
                sem)

    def drain(stage, sem):
        pltpu.make_async_copy(
            stage, out_hbm.at[0, 0, pl.ds(0, 128), pl.ds(0, 128)], sem).wait()

    def pair_body(p, _):
        rbA = row0 + p * 2 * NSHIFT

        @pl.when(p > 0)
        def _():
            drain(stageA, dsemA)

        build_and_issue(stageA, dsemA, rbA)

        @pl.when(p > 0)
        def _():
            drain(stageB, dsemB)

        build_and_issue(stageB, dsemB, rbA + NSHIFT)
        return 0

    lax.fori_loop(0, ROWS_PER_WORKER // (2 * NSHIFT), pair_body, 0)
    drain(stageA, dsemA)
    drain(stageB, dsemB)


def _gate_body(q_ref, wur_ref, wi_ref, scale_ref, out_ref):
    m = jnp.mean(q_ref[...], axis=0)
    gm = (m[:HEAD_DIM] + m[HEAD_DIM:]) * 0.5
    gr = 1.0 / (1.0 + jnp.exp(-(wur_ref[...] @ gm)))
    gu = 1.0 / (1.0 + jnp.exp(-(wi_ref[...] @ gm)))
    out_ref[...] = 1.0 + gu * scale_ref[...] * gr


@jax.jit
def _run(q, tabf, wur, wi, scale, bconst):
    one_plus_g = pl.pallas_call(
        _gate_body,
        out_shape=jax.ShapeDtypeStruct((NUM_HEADS,), jnp.float32),
    )(q, wur, wi, scale)

    mesh = plsc.VectorSubcoreMesh(core_axis_name="c", subcore_axis_name="s")
    kfn = functools.partial(
        pl.kernel,
        mesh=mesh,
        compiler_params=pltpu.CompilerParams(needs_layout_passes=False),
        out_type=jax.ShapeDtypeStruct((1, NUM_HEADS, T, T), jnp.float32),
        scratch_types=[
            pltpu.VMEM((NUM_BUCKETS * NUM_HEADS,), jnp.float32),
            pltpu.VMEM((VLEN,), jnp.int32),
            pltpu.VMEM((NUM_HEADS,), jnp.float32),
            pltpu.VMEM((VROW + 48,), jnp.float32),
            pltpu.VMEM((128, 128), jnp.float32),
            pltpu.VMEM((128, 128), jnp.float32),
            pltpu.SemaphoreType.DMA,
            pltpu.SemaphoreType.DMA,
        ],
    )(_sc_body)
    return kfn(tabf, bconst, one_plus_g)


def kernel(q, rel_pos_table, W_ur, W_i, scale, seq_len):
    q2 = q.reshape(QROWS // 2, 2 * HEAD_DIM)
    tabf = rel_pos_table.reshape(-1)
    bconst = jnp.asarray(_BUCKETS)
    return _run(q2, tabf, W_ur, W_i, scale, bconst)


# speedup vs baseline: 1.0411x; 1.0411x over previous
import functools
import math

import numpy as np
import jax
import jax.numpy as jnp
from jax import lax
from jax.experimental import pallas as pl
from jax.experimental.pallas import tpu as pltpu
from jax.experimental.pallas import tpu_sc as plsc

NUM_HEADS = 16
HEAD_DIM = 64
NUM_BUCKETS = 320
MAX_DISTANCE = 800
T = 2048
VLEN = 2 * T + 16
VROW = 2 * T
NSHIFT = 8
L = 16

QROWS = NUM_HEADS * T
QCHUNK = 128
ROWS_PER_WORKER = T // 2


def _bucket_vector() -> np.ndarray:
    d = np.arange(VLEN, dtype=np.int64) - (T - 1)
    half = NUM_BUCKETS // 2
    threshold = half // 2
    sign = (d >= 0).astype(np.int64)
    a = np.abs(d)
    log_ratio = np.log(np.clip(a.astype(np.float32), 1.0, None) / np.float32(threshold)) \
        / np.float32(math.log(MAX_DISTANCE / threshold))
    log_pos = np.minimum(
        (np.float32(threshold) + log_ratio * np.float32(half - threshold)).astype(np.int64),
        half - 1)
    b = np.where(a < threshold, a, log_pos) + sign * half
    b = np.clip(b, 0, NUM_BUCKETS - 1).astype(np.int32)
    b[2 * T - 1:] = b[2 * T - 2]
    return b


_BUCKETS = _bucket_vector()


def _sc_body(tab_hbm, b_hbm, g_hbm, out_hbm,
             tabv, bvec, gvec, vvec, stageA, stageB, dsemA, dsemB):
    c = lax.axis_index("c")
    s = lax.axis_index("s")
    h = s
    lanes = jnp.arange(L, dtype=jnp.int32)

    pltpu.sync_copy(tab_hbm, tabv)
    pltpu.sync_copy(b_hbm, bvec)
    pltpu.sync_copy(g_hbm, gvec)
    one_plus_g = plsc.load_gather(gvec, [jnp.full((L,), h, jnp.int32)])

    def v_body(i, _):
        m = i * L
        bidx = bvec[pl.ds(m, L)]
        vvec[pl.ds(m, L)] = plsc.load_gather(tabv, [bidx * NUM_HEADS + h]) * one_plus_g
        return 0

    lax.fori_loop(0, VROW // L, v_body, 0)

    row0 = c * ROWS_PER_WORKER

    def build_and_issue(stage, sem, rb):
        obase = (T - 1) - rb
        base = lanes + obase

        idxs = tuple(base - k for k in range(NSHIFT))
        vals = tuple(plsc.load_gather(vvec, [idxs[k]]) for k in range(NSHIFT))

        def t_body(t, carry):
            idxs, vals = carry
            new_idxs = tuple(idxs[k] + L for k in range(NSHIFT))
            new_vals = tuple(
                plsc.load_gather(vvec, [new_idxs[k]]) for k in range(NSHIFT))
            col = t * L
            for k in range(NSHIFT):
                stage[k, pl.ds(col, L)] = vals[k]
            return (new_idxs, new_vals)

        lax.fori_loop(0, T // L, t_body, (idxs, vals))

        pltpu.async_copy(
            stage,
            out_hbm.at[0, h, pl.ds(pl.multiple_of(rb, 8), NSHIFT), :],
            sem)

    def drain(stage, sem):
        pltpu.make_async_copy(
            stage, out_hbm.at[0, 0, pl.ds(0, NSHIFT), :], sem).wait()

    def pair_body(p, _):
        rbA = row0 + p * 2 * NSHIFT

        @pl.when(p > 0)
        def _():
            drain(stageA, dsemA)

        build_and_issue(stageA, dsemA, rbA)

        @pl.when(p > 0)
        def _():
            drain(stageB, dsemB)

        build_and_issue(stageB, dsemB, rbA + NSHIFT)
        return 0

    lax.fori_loop(0, ROWS_PER_WORKER // (2 * NSHIFT), pair_body, 0)
    drain(stageA, dsemA)
    drain(stageB, dsemB)


def _gate_body(q_ref, wur_ref, wi_ref, scale_ref, out_ref):
    gm = jnp.mean(q_ref[0], axis=(0, 1))
    gr = 1.0 / (1.0 + jnp.exp(-(wur_ref[...] @ gm)))
    gu = 1.0 / (1.0 + jnp.exp(-(wi_ref[...] @ gm)))
    out_ref[...] = 1.0 + gu * scale_ref[...] * gr


@jax.jit
def _run(q, tabf, wur, wi, scale, bconst):
    one_plus_g = pl.pallas_call(
        _gate_body,
        out_shape=jax.ShapeDtypeStruct((NUM_HEADS,), jnp.float32),
    )(q, wur, wi, scale)

    mesh = plsc.VectorSubcoreMesh(core_axis_name="c", subcore_axis_name="s")
    kfn = functools.partial(
        pl.kernel,
        mesh=mesh,
        compiler_params=pltpu.CompilerParams(needs_layout_passes=False),
        out_type=jax.ShapeDtypeStruct((1, NUM_HEADS, T, T), jnp.float32),
        scratch_types=[
            pltpu.VMEM((NUM_BUCKETS * NUM_HEADS,), jnp.float32),
            pltpu.VMEM((VLEN,), jnp.int32),
            pltpu.VMEM((NUM_HEADS,), jnp.float32),
            pltpu.VMEM((VROW + 48,), jnp.float32),
            pltpu.VMEM((NSHIFT, T), jnp.float32),
            pltpu.VMEM((NSHIFT, T), jnp.float32),
            pltpu.SemaphoreType.DMA,
            pltpu.SemaphoreType.DMA,
        ],
    )(_sc_body)
    return kfn(tabf, bconst, one_plus_g)


def kernel(q, rel_pos_table, W_ur, W_i, scale, seq_len):
    tabf = rel_pos_table.reshape(-1)
    bconst = jnp.asarray(_BUCKETS)
    return _run(q, tabf, W_ur, W_i, scale, bconst)
